# baseline (speedup 1.0000x reference)
import functools

import jax
import jax.numpy as jnp
from jax import lax
from jax.experimental import pallas as pl
from jax.experimental.pallas import tpu as pltpu
from jax.experimental.pallas import tpu_sc as plsc

TOPK = 32
ROW_LEN = 32768
NUM_ROWS = 32 * 32
LANES = 16
NV_ROW = ROW_LEN // LANES
GROUP = 8
NGROUP = NV_ROW // GROUP
CAND_CAP = 8192
STATIC_CAND_VREGS = 8

NEG_INF = float("-inf")
BIG_I32 = 2**31 - 1
_IN_BOUNDS = "promise_in_bounds"


def _shuf(x, pv):
  return x.at[pv].get(mode=_IN_BOUNDS, unique_indices=True)


def _splat_max(v, lane):
  for d in (8, 4, 2, 1):
    v = jnp.maximum(v, _shuf(v, jnp.bitwise_xor(lane, d)))
  return v


def _splat_min(v, lane):
  for d in (8, 4, 2, 1):
    v = jnp.minimum(v, _shuf(v, jnp.bitwise_xor(lane, d)))
  return v


def _tree_max(vs):
  while len(vs) > 1:
    vs = [jnp.maximum(a, b) for a, b in zip(vs[::2], vs[1::2])]
  return vs[0]


def _tree_min(vs):
  while len(vs) > 1:
    vs = [jnp.minimum(a, b) for a, b in zip(vs[::2], vs[1::2])]
  return vs[0]


def _kernel_body(nw, in_hbm, out_hbm, row_a, row_b, gmax_v, dlist_v,
                 cnt_buf, slot_ids, slot_cnt, cand_v, cand_i, out_v,
                 sem_a, sem_b):
  rows_per_w = NUM_ROWS // nw
  info = plsc.get_sparse_core_info()
  nc = info.num_cores
  w = lax.axis_index("s") * nc + lax.axis_index("c")
  lane = lax.iota(jnp.int32, LANES)
  lane0 = lane == 0
  row0 = w * rows_per_w

  def process_row(r_local, row_v):
    ninf = jnp.full((LANES,), NEG_INF, jnp.float32)
    big = jnp.full((LANES,), BIG_I32, jnp.int32)
    zi = jnp.zeros((LANES,), jnp.int32)
    zf = jnp.zeros((LANES,), jnp.float32)

    for k in range(STATIC_CAND_VREGS):
      cand_v[pl.ds(k * LANES, LANES)] = ninf

    def p0(g, carry):
      a0, a1 = carry
      vs = [row_v[pl.ds((g * 2 * GROUP + k) * LANES, LANES)]
            for k in range(2 * GROUP)]
      t0 = _tree_max(vs[:GROUP])
      t1 = _tree_max(vs[GROUP:])
      gmax_v[pl.ds((2 * g) * LANES, LANES)] = t0
      gmax_v[pl.ds((2 * g + 1) * LANES, LANES)] = t1
      return jnp.maximum(a0, t0), jnp.maximum(a1, t1)

    a0, a1 = lax.fori_loop(0, NGROUP // 2, p0, (ninf, ninf))
    thr_s = _splat_min(jnp.minimum(a0, a1), lane)

    def p1a(j, _):
      g = gmax_v[pl.ds(j * LANES, LANES)]
      m = g >= thr_s
      cnt = plsc.all_reduce_population_count(m)
      plsc.store_compressed(dlist_v.at[pl.ds(j * LANES, LANES)],
                            j * LANES + lane, mask=m)
      plsc.store_scatter(cnt_buf, [zi + j], cnt, mask=lane0)
      return _

    lax.fori_loop(0, NGROUP, p1a, jnp.int32(0))

    def l2(k, dsp):
      c = cnt_buf[pl.ds(k * LANES, LANES)]
      m = c > 0
      mi = jnp.where(m, jnp.int32(1), jnp.int32(0))
      pos = dsp + plsc.cumsum(mi) - 1
      plsc.store_scatter(slot_ids, [pos], k * LANES + lane, mask=m)
      plsc.store_scatter(slot_cnt, [pos], c, mask=m)
      return dsp + plsc.all_reduce_population_count(m)

    dsp = lax.fori_loop(0, NGROUP // LANES, l2, zi)
    n_slots = dsp[0]

    def slot_body(s, ptr):
      jslot = slot_ids[pl.ds(s, LANES)][0]
      c = slot_cnt[pl.ds(s, LANES)][0]
      ids16 = dlist_v[pl.ds(jslot * LANES, LANES)]

      def inner(q, p):
        sel = 2 * q + lax.shift_right_logical(lane, 3)
        valid = sel < c
        e = jnp.bitwise_and(_shuf(ids16, jnp.minimum(sel, LANES - 1)),
                            NGROUP * LANES - 1)
        col_base = (lax.shift_right_logical(e, 4) * (GROUP * LANES)
                    + jnp.bitwise_and(e, LANES - 1))
        elem_idx = col_base + jnp.bitwise_and(lane, 7) * LANES
        v = plsc.load_gather(row_v, [elem_idx])
        m = jnp.logical_and(v >= thr_s, valid)
        cnt = plsc.all_reduce_population_count(m)
        pc = jnp.minimum(p, CAND_CAP)
        plsc.store_compressed(cand_v.at[pl.ds(pc, LANES)], v, mask=m)
        plsc.store_compressed(cand_i.at[pl.ds(pc, LANES)], elem_idx, mask=m)
        return p + cnt[0]

      return lax.fori_loop(0, lax.shift_right_logical(c + 1, 1), inner, ptr)

    n_cand = lax.fori_loop(0, n_slots, slot_body, jnp.int32(0))

    plsc.store_scatter(cand_v, [jnp.minimum(n_cand, CAND_CAP) + lane], ninf)
    nv = lax.shift_right_logical(n_cand + LANES - 1, 4)

    init = (zf, zf, zi, zi)

    def extract_static(_):
      def rnd(t, carry):
        v0, v1, i0, i1 = carry
        vs = [cand_v[pl.ds(k * LANES, LANES)]
              for k in range(STATIC_CAND_VREGS)]
        mx = _splat_max(_tree_max(vs), lane)
        idxms = []
        for k in range(STATIC_CAND_VREGS):
          ii = cand_i[pl.ds(k * LANES, LANES)]
          idxms.append(jnp.where(vs[k] == mx, ii, big))
        bi = _splat_min(_tree_min(idxms), lane)
        pms = [jnp.where(idxms[k] == bi, k * LANES + lane, big)
               for k in range(STATIC_CAND_VREGS)]
        p = _splat_min(_tree_min(pms), lane)
        plsc.store_scatter(cand_v, [p], ninf, mask=lane0)
        v0 = jnp.where(lane == t, mx, v0)
        v1 = jnp.where(lane == t - LANES, mx, v1)
        i0 = jnp.where(lane == t, bi, i0)
        i1 = jnp.where(lane == t - LANES, bi, i1)
        return v0, v1, i0, i1

      return lax.fori_loop(0, TOPK, rnd, init)

    def extract_dynamic(_):
      def rnd(t, carry):
        v0, v1, i0, i1 = carry

        def fmax(j, acc):
          return jnp.maximum(acc, cand_v[pl.ds(j * LANES, LANES)])

        mx = _splat_max(lax.fori_loop(0, nv, fmax, ninf), lane)

        def fbi(j, best):
          v = cand_v[pl.ds(j * LANES, LANES)]
          ii = cand_i[pl.ds(j * LANES, LANES)]
          return jnp.minimum(best, jnp.where(v == mx, ii, big))

        bi = _splat_min(lax.fori_loop(0, nv, fbi, big), lane)

        def fp(j, best):
          v = cand_v[pl.ds(j * LANES, LANES)]
          ii = cand_i[pl.ds(j * LANES, LANES)]
          hit = jnp.logical_and(v == mx, ii == bi)
          return jnp.minimum(best, jnp.where(hit, j * LANES + lane, big))

        p = _splat_min(lax.fori_loop(0, nv, fp, big), lane)
        plsc.store_scatter(cand_v, [p], ninf, mask=lane0)
        v0 = jnp.where(lane == t, mx, v0)
        v1 = jnp.where(lane == t - LANES, mx, v1)
        i0 = jnp.where(lane == t, bi, i0)
        i1 = jnp.where(lane == t - LANES, bi, i1)
        return v0, v1, i0, i1

      return lax.fori_loop(0, TOPK, rnd, init)

    def extract_fallback(_):
      plsc.store_scatter(row_v, [ROW_LEN + lane], ninf)

      def rnd(t, carry):
        v0, v1, i0, i1 = carry

        def fmax(j, acc):
          return jnp.maximum(acc, row_v[pl.ds(j * LANES, LANES)])

        mx = _splat_max(lax.fori_loop(0, NV_ROW + 1, fmax, ninf), lane)

        def fbi(j, best):
          v = row_v[pl.ds(j * LANES, LANES)]
          return jnp.minimum(best, jnp.where(v == mx, j * LANES + lane, big))

        bi = _splat_min(lax.fori_loop(0, NV_ROW + 1, fbi, big), lane)
        plsc.store_scatter(row_v, [bi], ninf, mask=lane0)
        v0 = jnp.where(lane == t, mx, v0)
        v1 = jnp.where(lane == t - LANES, mx, v1)
        i0 = jnp.where(lane == t, bi, i0)
        i1 = jnp.where(lane == t - LANES, bi, i1)
        return v0, v1, i0, i1

      return lax.fori_loop(0, TOPK, rnd, init)

    def extract_small(_):
      return lax.cond(n_cand <= STATIC_CAND_VREGS * LANES,
                      extract_static, extract_dynamic, 0)

    v0, v1, i0, i1 = lax.cond(n_cand <= CAND_CAP,
                              extract_small, extract_fallback, 0)

    k0, w0 = plsc.sort_key_val(i0, v0)
    k1, w1 = plsc.sort_key_val(i1, v1)
    rk = _shuf(k1, LANES - 1 - lane)
    rv = _shuf(w1, LANES - 1 - lane)
    cmp = k0 <= rk
    lok, lov = jnp.where(cmp, k0, rk), jnp.where(cmp, w0, rv)
    hik, hiv = jnp.where(cmp, rk, k0), jnp.where(cmp, rv, w0)
    for d in (8, 4, 2, 1):
      keep_small = jnp.bitwise_and(lane, d) == 0
      nlk, nlv, nhk, nhv = [], [], [], []
      pk = _shuf(lok, jnp.bitwise_xor(lane, d))
      pv2 = _shuf(lov, jnp.bitwise_xor(lane, d))
      nk = jnp.where(keep_small, jnp.minimum(lok, pk), jnp.maximum(lok, pk))
      lov = jnp.where(nk == lok, lov, pv2)
      lok = nk
      pk = _shuf(hik, jnp.bitwise_xor(lane, d))
      pv2 = _shuf(hiv, jnp.bitwise_xor(lane, d))
      nk = jnp.where(keep_small, jnp.minimum(hik, pk), jnp.maximum(hik, pk))
      hiv = jnp.where(nk == hik, hiv, pv2)
      hik = nk

    base = r_local * TOPK
    plsc.store_scatter(out_v, [base + lane], lov)
    plsc.store_scatter(out_v, [base + LANES + lane], hiv)

  pltpu.async_copy(in_hbm.at[row0], row_a.at[pl.ds(0, ROW_LEN)], sem_a)

  def pair_body(i, _):
    r_even = 2 * i
    pltpu.make_async_copy(in_hbm.at[row0], row_a.at[pl.ds(0, ROW_LEN)],
                          sem_a).wait()
    pltpu.async_copy(in_hbm.at[row0 + r_even + 1],
                     row_b.at[pl.ds(0, ROW_LEN)], sem_b)
    process_row(r_even, row_a)
    pltpu.make_async_copy(in_hbm.at[row0], row_b.at[pl.ds(0, ROW_LEN)],
                          sem_b).wait()

    @pl.when(r_even + 2 < rows_per_w)
    def _start_next():
      pltpu.async_copy(in_hbm.at[row0 + r_even + 2],
                       row_a.at[pl.ds(0, ROW_LEN)], sem_a)

    process_row(r_even + 1, row_b)
    return _

  lax.fori_loop(0, rows_per_w // 2, pair_body, jnp.int32(0))
  pltpu.sync_copy(out_v, out_hbm.at[pl.ds(w * rows_per_w * TOPK,
                                          rows_per_w * TOPK)])


def kernel(inputs):
  info = plsc.get_sparse_core_info()
  nw = info.num_cores * info.num_subcores
  rows_per_w = NUM_ROWS // nw
  mesh = plsc.VectorSubcoreMesh(core_axis_name="c", subcore_axis_name="s")
  k = pl.kernel(
      functools.partial(_kernel_body, nw),
      out_type=jax.ShapeDtypeStruct((NUM_ROWS * TOPK,), jnp.float32),
      mesh=mesh,
      compiler_params=pltpu.CompilerParams(needs_layout_passes=False),
      scratch_types=[
          pltpu.VMEM((ROW_LEN + LANES,), jnp.float32),
          pltpu.VMEM((ROW_LEN + LANES,), jnp.float32),
          pltpu.VMEM((NGROUP * LANES,), jnp.float32),
          pltpu.VMEM((NGROUP * LANES,), jnp.int32),
          pltpu.VMEM((NGROUP,), jnp.int32),
          pltpu.VMEM((NGROUP + LANES,), jnp.int32),
          pltpu.VMEM((NGROUP + LANES,), jnp.int32),
          pltpu.VMEM((CAND_CAP + LANES,), jnp.float32),
          pltpu.VMEM((CAND_CAP + LANES,), jnp.int32),
          pltpu.VMEM((rows_per_w * TOPK,), jnp.float32),
          pltpu.SemaphoreType.DMA,
          pltpu.SemaphoreType.DMA,
      ],
  )
  out = k(inputs.reshape(NUM_ROWS, ROW_LEN))
  return out.reshape(32, 32, TOPK)


# speedup vs baseline: 1.3865x; 1.3865x over previous
import functools

import jax
import jax.numpy as jnp
from jax import lax
from jax.experimental import pallas as pl
from jax.experimental.pallas import tpu as pltpu
from jax.experimental.pallas import tpu_sc as plsc

TOPK = 32
ROW_LEN = 32768
NUM_ROWS = 32 * 32
LANES = 16
NV_ROW = ROW_LEN // LANES
GROUP = 8
NGROUP = NV_ROW // GROUP
CAND_CAP = 8192
STATIC_CAND_VREGS = 8

NEG_INF = float("-inf")
BIG_I32 = 2**31 - 1
_IN_BOUNDS = "promise_in_bounds"


def _shuf(x, pv):
  return x.at[pv].get(mode=_IN_BOUNDS, unique_indices=True)


def _splat_max(v, lane):
  for d in (8, 4, 2, 1):
    v = jnp.maximum(v, _shuf(v, jnp.bitwise_xor(lane, d)))
  return v


def _splat_min(v, lane):
  for d in (8, 4, 2, 1):
    v = jnp.minimum(v, _shuf(v, jnp.bitwise_xor(lane, d)))
  return v


def _bitonic_clean_desc(x, lane):
  for d in (8, 4, 2, 1):
    p = _shuf(x, jnp.bitwise_xor(lane, d))
    x = jnp.where(jnp.bitwise_and(lane, d) == 0, jnp.maximum(x, p),
                  jnp.minimum(x, p))
  return x


def _merge16_hi(a, b, lane):
  rb = _shuf(b, 15 - lane)
  return _bitonic_clean_desc(jnp.maximum(a, rb), lane)


def _merge16_full(a, b, lane):
  rb = _shuf(b, 15 - lane)
  hi = _bitonic_clean_desc(jnp.maximum(a, rb), lane)
  lo = _bitonic_clean_desc(jnp.minimum(a, rb), lane)
  return hi, lo


def _tree_max(vs):
  while len(vs) > 1:
    vs = [jnp.maximum(a, b) for a, b in zip(vs[::2], vs[1::2])]
  return vs[0]


def _tree_min(vs):
  while len(vs) > 1:
    vs = [jnp.minimum(a, b) for a, b in zip(vs[::2], vs[1::2])]
  return vs[0]


def _kernel_body(nw, in_hbm, out_hbm, row_a, row_b, gmax_v, dlist_v,
                 cnt_buf, slot_ids, slot_cnt, cand_v, cand_i, out_v,
                 sem_a, sem_b):
  rows_per_w = NUM_ROWS // nw
  info = plsc.get_sparse_core_info()
  nc = info.num_cores
  w = lax.axis_index("s") * nc + lax.axis_index("c")
  lane = lax.iota(jnp.int32, LANES)
  lane0 = lane == 0
  row0 = w * rows_per_w

  def process_row(r_local, row_v):
    ninf = jnp.full((LANES,), NEG_INF, jnp.float32)
    big = jnp.full((LANES,), BIG_I32, jnp.int32)
    zi = jnp.zeros((LANES,), jnp.int32)
    zf = jnp.zeros((LANES,), jnp.float32)

    for k in range(STATIC_CAND_VREGS):
      cand_v[pl.ds(k * LANES, LANES)] = ninf

    def p0(g, carry):
      a0, a1 = carry
      vs = [row_v[pl.ds((g * 2 * GROUP + k) * LANES, LANES)]
            for k in range(2 * GROUP)]
      t0 = _tree_max(vs[:GROUP])
      t1 = _tree_max(vs[GROUP:])
      gmax_v[pl.ds((2 * g) * LANES, LANES)] = t0
      gmax_v[pl.ds((2 * g + 1) * LANES, LANES)] = t1
      return jnp.maximum(a0, t0), jnp.maximum(a1, t1)

    a0, a1 = lax.fori_loop(0, NGROUP // 2, p0, (ninf, ninf))
    thr_s = _splat_min(jnp.minimum(a0, a1), lane)

    def p1a(j, _):
      g = gmax_v[pl.ds(j * LANES, LANES)]
      m = g >= thr_s
      cnt = plsc.all_reduce_population_count(m)
      plsc.store_compressed(dlist_v.at[pl.ds(j * LANES, LANES)],
                            j * LANES + lane, mask=m)
      plsc.store_scatter(cnt_buf, [zi + j], cnt, mask=lane0)
      return _

    lax.fori_loop(0, NGROUP, p1a, jnp.int32(0))

    def l2(k, dsp):
      c = cnt_buf[pl.ds(k * LANES, LANES)]
      m = c > 0
      mi = jnp.where(m, jnp.int32(1), jnp.int32(0))
      pos = dsp + plsc.cumsum(mi) - 1
      plsc.store_scatter(slot_ids, [pos], (k * LANES + lane) * 32 + c, mask=m)
      return dsp + plsc.all_reduce_population_count(m)

    dsp = lax.fori_loop(0, NGROUP // LANES, l2, zi)
    n_slots = dsp[0]

    def slot_body(s, ptr):
      rec = slot_ids[pl.ds(s, LANES)][0]
      jslot = lax.shift_right_logical(rec, 5)
      c = jnp.bitwise_and(rec, 31)
      ids16 = dlist_v[pl.ds(jslot * LANES, LANES)]

      def inner(q, p):
        sel = 2 * q + lax.shift_right_logical(lane, 3)
        valid = sel < c
        e = jnp.bitwise_and(_shuf(ids16, jnp.minimum(sel, LANES - 1)),
                            NGROUP * LANES - 1)
        col_base = (lax.shift_right_logical(e, 4) * (GROUP * LANES)
                    + jnp.bitwise_and(e, LANES - 1))
        elem_idx = col_base + jnp.bitwise_and(lane, 7) * LANES
        v = plsc.load_gather(row_v, [elem_idx])
        m = jnp.logical_and(v >= thr_s, valid)
        cnt = plsc.all_reduce_population_count(m)
        pc = jnp.minimum(p, CAND_CAP)
        plsc.store_compressed(cand_v.at[pl.ds(pc, LANES)], v, mask=m)
        plsc.store_compressed(cand_i.at[pl.ds(pc, LANES)], elem_idx, mask=m)
        return p + cnt[0]

      return lax.fori_loop(0, lax.shift_right_logical(c + 1, 1), inner, ptr)

    n_cand = lax.fori_loop(0, n_slots, slot_body, jnp.int32(0))

    plsc.store_scatter(cand_v, [jnp.minimum(n_cand, CAND_CAP) + lane], ninf)
    nv = lax.shift_right_logical(n_cand + LANES - 1, 4)

    init = (zf, zf, zi, zi)

    def extract_static(_):
      def rnd(t, carry):
        v0, v1, i0, i1 = carry
        vs = [cand_v[pl.ds(k * LANES, LANES)]
              for k in range(STATIC_CAND_VREGS)]
        mx = _splat_max(_tree_max(vs), lane)
        idxms = []
        for k in range(STATIC_CAND_VREGS):
          ii = cand_i[pl.ds(k * LANES, LANES)]
          idxms.append(jnp.where(vs[k] == mx, ii, big))
        bi = _splat_min(_tree_min(idxms), lane)
        pms = [jnp.where(idxms[k] == bi, k * LANES + lane, big)
               for k in range(STATIC_CAND_VREGS)]
        p = _splat_min(_tree_min(pms), lane)
        plsc.store_scatter(cand_v, [p], ninf, mask=lane0)
        v0 = jnp.where(lane == t, mx, v0)
        v1 = jnp.where(lane == t - LANES, mx, v1)
        i0 = jnp.where(lane == t, bi, i0)
        i1 = jnp.where(lane == t - LANES, bi, i1)
        return v0, v1, i0, i1

      return lax.fori_loop(0, TOPK, rnd, init)

    def extract_dynamic(_):
      def rnd(t, carry):
        v0, v1, i0, i1 = carry

        def fmax(j, acc):
          return jnp.maximum(acc, cand_v[pl.ds(j * LANES, LANES)])

        mx = _splat_max(lax.fori_loop(0, nv, fmax, ninf), lane)

        def fbi(j, best):
          v = cand_v[pl.ds(j * LANES, LANES)]
          ii = cand_i[pl.ds(j * LANES, LANES)]
          return jnp.minimum(best, jnp.where(v == mx, ii, big))

        bi = _splat_min(lax.fori_loop(0, nv, fbi, big), lane)

        def fp(j, best):
          v = cand_v[pl.ds(j * LANES, LANES)]
          ii = cand_i[pl.ds(j * LANES, LANES)]
          hit = jnp.logical_and(v == mx, ii == bi)
          return jnp.minimum(best, jnp.where(hit, j * LANES + lane, big))

        p = _splat_min(lax.fori_loop(0, nv, fp, big), lane)
        plsc.store_scatter(cand_v, [p], ninf, mask=lane0)
        v0 = jnp.where(lane == t, mx, v0)
        v1 = jnp.where(lane == t - LANES, mx, v1)
        i0 = jnp.where(lane == t, bi, i0)
        i1 = jnp.where(lane == t - LANES, bi, i1)
        return v0, v1, i0, i1

      return lax.fori_loop(0, TOPK, rnd, init)

    def extract_fallback(_):
      plsc.store_scatter(row_v, [ROW_LEN + lane], ninf)

      def rnd(t, carry):
        v0, v1, i0, i1 = carry

        def fmax(j, acc):
          return jnp.maximum(acc, row_v[pl.ds(j * LANES, LANES)])

        mx = _splat_max(lax.fori_loop(0, NV_ROW + 1, fmax, ninf), lane)

        def fbi(j, best):
          v = row_v[pl.ds(j * LANES, LANES)]
          return jnp.minimum(best, jnp.where(v == mx, j * LANES + lane, big))

        bi = _splat_min(lax.fori_loop(0, NV_ROW + 1, fbi, big), lane)
        plsc.store_scatter(row_v, [bi], ninf, mask=lane0)
        v0 = jnp.where(lane == t, mx, v0)
        v1 = jnp.where(lane == t - LANES, mx, v1)
        i0 = jnp.where(lane == t, bi, i0)
        i1 = jnp.where(lane == t - LANES, bi, i1)
        return v0, v1, i0, i1

      return lax.fori_loop(0, TOPK, rnd, init)

    def extract_small(_):
      return lax.cond(n_cand <= STATIC_CAND_VREGS * LANES,
                      extract_static, extract_dynamic, 0)

    nv_c = jnp.minimum(nv, CAND_CAP // LANES)

    def vmerge(j, carry):
      s0, s1 = carry
      c = plsc.sort_key_val(cand_v[pl.ds(j * LANES, LANES)], zi,
                            descending=True)[0]
      m_hi = _merge16_hi(s1, c, lane)
      return _merge16_full(s0, m_hi, lane)

    s0, s1 = lax.fori_loop(0, nv_c, vmerge, (ninf, ninf))
    t32 = _shuf(s1, zi + 15)

    def wcompact(j, wp):
      v = cand_v[pl.ds(j * LANES, LANES)]
      ii = cand_i[pl.ds(j * LANES, LANES)]
      m = v >= t32
      cnt = plsc.all_reduce_population_count(m)
      pc = jnp.minimum(wp, NGROUP * LANES - LANES)
      plsc.store_compressed(gmax_v.at[pl.ds(pc, LANES)], v, mask=m)
      plsc.store_compressed(dlist_v.at[pl.ds(pc, LANES)], ii, mask=m)
      return wp + cnt[0]

    n_win = lax.fori_loop(0, nv_c, wcompact, jnp.int32(0))

    def fast_finish(_):
      return (gmax_v[pl.ds(0, LANES)], gmax_v[pl.ds(LANES, LANES)],
              dlist_v[pl.ds(0, LANES)], dlist_v[pl.ds(LANES, LANES)])

    def slow(_):
      return lax.cond(n_cand <= CAND_CAP, extract_small, extract_fallback, 0)

    v0, v1, i0, i1 = lax.cond(
        jnp.logical_and(n_win == TOPK, n_cand <= CAND_CAP),
        fast_finish, slow, 0)

    k0, w0 = plsc.sort_key_val(i0, v0)
    k1, w1 = plsc.sort_key_val(i1, v1)
    rk = _shuf(k1, LANES - 1 - lane)
    rv = _shuf(w1, LANES - 1 - lane)
    cmp = k0 <= rk
    lok, lov = jnp.where(cmp, k0, rk), jnp.where(cmp, w0, rv)
    hik, hiv = jnp.where(cmp, rk, k0), jnp.where(cmp, rv, w0)
    for d in (8, 4, 2, 1):
      keep_small = jnp.bitwise_and(lane, d) == 0
      nlk, nlv, nhk, nhv = [], [], [], []
      pk = _shuf(lok, jnp.bitwise_xor(lane, d))
      pv2 = _shuf(lov, jnp.bitwise_xor(lane, d))
      nk = jnp.where(keep_small, jnp.minimum(lok, pk), jnp.maximum(lok, pk))
      lov = jnp.where(nk == lok, lov, pv2)
      lok = nk
      pk = _shuf(hik, jnp.bitwise_xor(lane, d))
      pv2 = _shuf(hiv, jnp.bitwise_xor(lane, d))
      nk = jnp.where(keep_small, jnp.minimum(hik, pk), jnp.maximum(hik, pk))
      hiv = jnp.where(nk == hik, hiv, pv2)
      hik = nk

    base = r_local * TOPK
    plsc.store_scatter(out_v, [base + lane], lov)
    plsc.store_scatter(out_v, [base + LANES + lane], hiv)

  pltpu.async_copy(in_hbm.at[row0], row_a.at[pl.ds(0, ROW_LEN)], sem_a)

  def pair_body(i, _):
    r_even = 2 * i
    pltpu.make_async_copy(in_hbm.at[row0], row_a.at[pl.ds(0, ROW_LEN)],
                          sem_a).wait()
    pltpu.async_copy(in_hbm.at[row0 + r_even + 1],
                     row_b.at[pl.ds(0, ROW_LEN)], sem_b)
    process_row(r_even, row_a)
    pltpu.make_async_copy(in_hbm.at[row0], row_b.at[pl.ds(0, ROW_LEN)],
                          sem_b).wait()

    @pl.when(r_even + 2 < rows_per_w)
    def _start_next():
      pltpu.async_copy(in_hbm.at[row0 + r_even + 2],
                       row_a.at[pl.ds(0, ROW_LEN)], sem_a)

    process_row(r_even + 1, row_b)
    return _

  lax.fori_loop(0, rows_per_w // 2, pair_body, jnp.int32(0))
  pltpu.sync_copy(out_v, out_hbm.at[pl.ds(w * rows_per_w * TOPK,
                                          rows_per_w * TOPK)])


def kernel(inputs):
  info = plsc.get_sparse_core_info()
  nw = info.num_cores * info.num_subcores
  rows_per_w = NUM_ROWS // nw
  mesh = plsc.VectorSubcoreMesh(core_axis_name="c", subcore_axis_name="s")
  k = pl.kernel(
      functools.partial(_kernel_body, nw),
      out_type=jax.ShapeDtypeStruct((NUM_ROWS * TOPK,), jnp.float32),
      mesh=mesh,
      compiler_params=pltpu.CompilerParams(needs_layout_passes=False),
      scratch_types=[
          pltpu.VMEM((ROW_LEN + LANES,), jnp.float32),
          pltpu.VMEM((ROW_LEN + LANES,), jnp.float32),
          pltpu.VMEM((NGROUP * LANES,), jnp.float32),
          pltpu.VMEM((NGROUP * LANES,), jnp.int32),
          pltpu.VMEM((NGROUP,), jnp.int32),
          pltpu.VMEM((NGROUP + LANES,), jnp.int32),
          pltpu.VMEM((NGROUP + LANES,), jnp.int32),
          pltpu.VMEM((CAND_CAP + LANES,), jnp.float32),
          pltpu.VMEM((CAND_CAP + LANES,), jnp.int32),
          pltpu.VMEM((rows_per_w * TOPK,), jnp.float32),
          pltpu.SemaphoreType.DMA,
          pltpu.SemaphoreType.DMA,
      ],
  )
  out = k(inputs.reshape(NUM_ROWS, ROW_LEN))
  return out.reshape(32, 32, TOPK)
